# baseline (speedup 1.0000x reference)
import functools

import jax
import jax.numpy as jnp
from jax import lax
from jax.experimental import pallas as pl
from jax.experimental.pallas import tpu as pltpu
from jax.experimental.pallas import tpu_sc as plsc

DIM = 32
NC, NS = 2, 16
NW = NC * NS
BW = 128
SEQ = 200
BBLK = 4096 // BW
UNITS = SEQ * BBLK
UPW = UNITS // NW


@functools.lru_cache(maxsize=None)
def _build():
    mesh = plsc.VectorSubcoreMesh(core_axis_name="c", subcore_axis_name="s")

    def body(x_u, t_rm, out_u, idx_all, buf0, buf1, bt0, bt1,
             gs0, gs1, os0, os1):
        bufs, bts = (buf0, buf1), (bt0, bt1)
        gsems, osems = (gs0, gs1), (os0, os1)
        wid = lax.axis_index("s") * NC + lax.axis_index("c")
        u0 = wid * UPW

        pltpu.sync_copy(x_u.at[pl.ds(u0, UPW)], idx_all)
        lane = lax.iota(jnp.int32, 16)

        def fire_gather(u, b):
            pltpu.async_copy(t_rm.at[idx_all.at[u]], bufs[b], gsems[b])

        def drain_gather(u, b):
            pltpu.make_async_copy(
                t_rm.at[idx_all.at[u]], bufs[b], gsems[b]).wait()

        def transpose(b):
            def tr_d(d, carry):
                cols = jnp.full((16,), 0, jnp.int32) + d
                for k0 in range(0, BW, 16):
                    v = plsc.load_gather(bufs[b], [lane + k0, cols])
                    bts[b][pl.ds(d * BW + k0, 16)] = v
                return carry
            lax.fori_loop(0, DIM, tr_d, 0)

        def fire_writes(u, b):
            gu = u0 + u
            s = gu // BBLK
            bb = gu % BBLK
            for db in range(4):
                pltpu.async_copy(
                    bts[b].at[pl.ds(db * 1024, 1024)],
                    out_u.at[s * 128 + db * 32 + bb],
                    osems[b])

        def drain_writes(u, b):
            gu = u0 + u
            s = gu // BBLK
            bb = gu % BBLK
            for db in range(4):
                pltpu.make_async_copy(
                    bts[b].at[pl.ds(db * 1024, 1024)],
                    out_u.at[s * 128 + db * 32 + bb],
                    osems[b]).wait()

        for b in range(2):
            fire_gather(b, b)
        for u in range(2):
            b = u
            drain_gather(u, b)
            transpose(b)
            fire_writes(u, b)
            fire_gather(u + 2, b)

        def step_fn(it, carry):
            for b in range(2):
                u = it * 2 + b
                drain_gather(u, b)
                drain_writes(u - 2, b)
                transpose(b)
                fire_writes(u, b)
                fire_gather(u + 2, b)
            return carry

        lax.fori_loop(1, (UPW - 2) // 2, step_fn, 0)

        for u in range(UPW - 2, UPW):
            b = u % 2
            drain_gather(u, b)
            drain_writes(u - 2, b)
            transpose(b)
            fire_writes(u, b)
        for u in range(UPW - 2, UPW):
            drain_writes(u, u % 2)

    return pl.kernel(
        body,
        out_type=jax.ShapeDtypeStruct((UNITS * 4, 1024), jnp.float32),
        mesh=mesh,
        scratch_types=[
            pltpu.VMEM((UPW, BW), jnp.int32),
            pltpu.VMEM((BW, DIM), jnp.float32),
            pltpu.VMEM((BW, DIM), jnp.float32),
            pltpu.VMEM((BW * DIM,), jnp.float32),
            pltpu.VMEM((BW * DIM,), jnp.float32),
            pltpu.SemaphoreType.DMA,
            pltpu.SemaphoreType.DMA,
            pltpu.SemaphoreType.DMA,
            pltpu.SemaphoreType.DMA,
        ],
        compiler_params=pltpu.CompilerParams(
            use_tc_tiling_on_sc=False, needs_layout_passes=False),
    )


@jax.jit
def kernel(x, table):
    b, s = x.shape
    x_u = jnp.transpose(x).reshape(UNITS, BW).astype(jnp.int32)
    out_u = _build()(x_u, table)
    y = out_u.reshape(SEQ, 4, BBLK, 8, BW).transpose(2, 4, 0, 1, 3)
    return y.reshape(b, s, DIM)
